# Initial kernel scaffold; baseline (speedup 1.0000x reference)
import jax
import jax.numpy as jnp
from jax.experimental import pallas as pl


def kernel(hidden_states, router_w, gate_up_w, down_w, shared_gate_up_w, shared_down_w, shared_gate_w):
    raise NotImplementedError("write your pallas kernel here")


# speedup vs baseline: 1.6677x; 1.6677x over previous
import functools

import jax
import jax.numpy as jnp
from jax import lax
from jax.experimental import pallas as pl
from jax.experimental.pallas import tpu as pltpu
from jax.experimental.pallas import tpu_sc as plsc

D_MODEL = 2048
D_FF = 1408
NUM_E = 8
T = 2048
TOPK = 2

BM = 128
NT = 40
NPAD = NT * BM
TB = 256
NTE = 64

NW = 32
CH = 16
NCHUNK = T // NW // CH
CB = 128


def _router_body(x_ref, rw_ref, sgw_ref, i1_ref, i2_ref, h1_ref, h2_ref, sg_ref):
    x = x_ref[...]
    rw = rw_ref[...]
    logits = jnp.dot(x, rw, preferred_element_type=jnp.float32)
    ei = lax.broadcasted_iota(jnp.int32, (TB, NUM_E), 1)
    m1 = jnp.max(logits, axis=1, keepdims=True)
    i1 = jnp.min(jnp.where(logits == m1, ei, NUM_E), axis=1, keepdims=True)
    masked = jnp.where(ei == i1, -jnp.inf, logits)
    m2 = jnp.max(masked, axis=1, keepdims=True)
    i2 = jnp.min(jnp.where(masked == m2, ei, NUM_E), axis=1, keepdims=True)
    w1 = jax.nn.sigmoid(m1 - m2)
    w2 = jax.nn.sigmoid(m2 - m1)
    i1_ref[...] = i1
    i2_ref[...] = i2
    h1_ref[...] = x * w1
    h2_ref[...] = x * w2
    sgl = jnp.sum(x * sgw_ref[...], axis=1, keepdims=True)
    sg_ref[...] = jax.nn.sigmoid(sgl)


def _router(hidden, router_w, sgw_row):
    return pl.pallas_call(
        _router_body,
        grid=(T // TB,),
        in_specs=[
            pl.BlockSpec((TB, D_MODEL), lambda i: (i, 0)),
            pl.BlockSpec((D_MODEL, NUM_E), lambda i: (0, 0)),
            pl.BlockSpec((1, D_MODEL), lambda i: (0, 0)),
        ],
        out_specs=[
            pl.BlockSpec((TB, 1), lambda i: (i, 0)),
            pl.BlockSpec((TB, 1), lambda i: (i, 0)),
            pl.BlockSpec((TB, D_MODEL), lambda i: (i, 0)),
            pl.BlockSpec((TB, D_MODEL), lambda i: (i, 0)),
            pl.BlockSpec((TB, 1), lambda i: (i, 0)),
        ],
        out_shape=[
            jax.ShapeDtypeStruct((T, 1), jnp.int32),
            jax.ShapeDtypeStruct((T, 1), jnp.int32),
            jax.ShapeDtypeStruct((T, D_MODEL), jnp.float32),
            jax.ShapeDtypeStruct((T, D_MODEL), jnp.float32),
            jax.ShapeDtypeStruct((T, 1), jnp.float32),
        ],
    )(hidden, router_w, sgw_row)


def _sortmeta_body(i1_ref, i2_ref, p1_ref, p2_ref, te_ref, s_ref):
    ei_blk = lax.broadcasted_iota(jnp.int32, (CB, NUM_E), 1)
    lexc = (lax.broadcasted_iota(jnp.int32, (CB, CB), 0)
            > lax.broadcasted_iota(jnp.int32, (CB, CB), 1)).astype(jnp.bfloat16)
    carry = jnp.zeros((1, NUM_E), jnp.float32)
    for b in range(T // CB):
        i1b = i1_ref[pl.ds(b * CB, CB), :]
        i2b = i2_ref[pl.ds(b * CB, CB), :]
        ob = ((i1b == ei_blk) | (i2b == ei_blk)).astype(jnp.float32)
        c = jnp.dot(lexc, ob.astype(jnp.bfloat16), preferred_element_type=jnp.float32)
        s_ref[pl.ds(b * CB, CB), :] = c + carry
        carry = carry + jnp.sum(ob, axis=0, keepdims=True)

    counts = carry.astype(jnp.int32)
    tiles = (counts + (BM - 1)) // BM
    cum = tiles
    for sh in (1, 2, 4):
        cum = cum + jnp.concatenate(
            [jnp.zeros((1, sh), jnp.int32), cum[:, :NUM_E - sh]], axis=1)
    off_slots = (cum - tiles) * BM

    ei = lax.broadcasted_iota(jnp.int32, (T, NUM_E), 1)
    so = s_ref[...].astype(jnp.int32) + off_slots
    o1 = (i1_ref[...] == ei).astype(jnp.int32)
    o2 = (i2_ref[...] == ei).astype(jnp.int32)
    p1_ref[...] = jnp.sum(o1 * so, axis=1, keepdims=True)
    p2_ref[...] = jnp.sum(o2 * so, axis=1, keepdims=True)

    jj = lax.broadcasted_iota(jnp.int32, (NTE, NUM_E), 0)
    te = jnp.sum((jj >= cum).astype(jnp.int32), axis=1, keepdims=True)
    te_ref[...] = jnp.minimum(te, NUM_E - 1)


def _sortmeta(i1, i2):
    return pl.pallas_call(
        _sortmeta_body,
        grid=(1,),
        in_specs=[
            pl.BlockSpec((T, 1), lambda i: (0, 0)),
            pl.BlockSpec((T, 1), lambda i: (0, 0)),
        ],
        out_specs=[
            pl.BlockSpec((T, 1), lambda i: (0, 0)),
            pl.BlockSpec((T, 1), lambda i: (0, 0)),
            pl.BlockSpec((NTE, 1), lambda i: (0, 0)),
        ],
        out_shape=[
            jax.ShapeDtypeStruct((T, 1), jnp.int32),
            jax.ShapeDtypeStruct((T, 1), jnp.int32),
            jax.ShapeDtypeStruct((NTE, 1), jnp.int32),
        ],
        scratch_shapes=[pltpu.VMEM((T, NUM_E), jnp.float32)],
    )(i1, i2)


def _sc_dispatch(h1, h2, p1r, p2r):
    mesh = plsc.VectorSubcoreMesh(core_axis_name="c", subcore_axis_name="s")

    @functools.partial(
        pl.kernel,
        out_type=jax.ShapeDtypeStruct((NPAD, D_MODEL), jnp.float32),
        mesh=mesh,
        scratch_types=[
            pltpu.VMEM((NCHUNK, CH), jnp.int32),
            pltpu.VMEM((CH, D_MODEL), jnp.float32),
            pltpu.SemaphoreType.DMA,
        ],
    )
    def k(h1_hbm, h2_hbm, p1_hbm, p2_hbm, xs_hbm, idx_v, rows_v, sem):
        wid = lax.axis_index("c") * 16 + lax.axis_index("s")
        base_i = wid * NCHUNK
        for src_hbm, p_hbm in ((h1_hbm, p1_hbm), (h2_hbm, p2_hbm)):
            pltpu.sync_copy(p_hbm.at[pl.ds(base_i, NCHUNK)], idx_v)
            for c in range(NCHUNK):
                pltpu.sync_copy(src_hbm.at[pl.ds((base_i + c) * CH, CH)], rows_v)
                pltpu.async_copy(rows_v, xs_hbm.at[idx_v.at[c]], sem).wait()

    return k(h1, h2, p1r, p2r)


def _sc_combine_fetch(ys, p1r, p2r):
    mesh = plsc.VectorSubcoreMesh(core_axis_name="c", subcore_axis_name="s")

    @functools.partial(
        pl.kernel,
        out_type=[
            jax.ShapeDtypeStruct((T, D_MODEL), jnp.float32),
            jax.ShapeDtypeStruct((T, D_MODEL), jnp.float32),
        ],
        mesh=mesh,
        scratch_types=[
            pltpu.VMEM((NCHUNK, CH), jnp.int32),
            pltpu.VMEM((CH, D_MODEL), jnp.float32),
            pltpu.SemaphoreType.DMA,
        ],
    )
    def k(ys_hbm, p1_hbm, p2_hbm, g0_hbm, g1_hbm, idx_v, rows_v, sem):
        wid = lax.axis_index("c") * 16 + lax.axis_index("s")
        base_i = wid * NCHUNK
        for p_hbm, o_hbm in ((p1_hbm, g0_hbm), (p2_hbm, g1_hbm)):
            pltpu.sync_copy(p_hbm.at[pl.ds(base_i, NCHUNK)], idx_v)
            for c in range(NCHUNK):
                pltpu.async_copy(ys_hbm.at[idx_v.at[c]], rows_v, sem).wait()
                pltpu.sync_copy(rows_v, o_hbm.at[pl.ds((base_i + c) * CH, CH)])

    return k(ys, p1r, p2r)


def _gu_body(te_ref, x_ref, w_ref, act_ref):
    del te_ref
    gu = jnp.dot(x_ref[...], w_ref[0], preferred_element_type=jnp.float32)
    g = gu[:, :D_FF]
    u = gu[:, D_FF:]
    act_ref[...] = g * jax.nn.sigmoid(g) * u


def _grouped_gu(te, xs, gup):
    gs = pltpu.PrefetchScalarGridSpec(
        num_scalar_prefetch=1,
        grid=(NT,),
        in_specs=[
            pl.BlockSpec((BM, D_MODEL), lambda j, te_ref: (j, 0)),
            pl.BlockSpec((1, D_MODEL, 2 * D_FF), lambda j, te_ref: (te_ref[j], 0, 0)),
        ],
        out_specs=pl.BlockSpec((BM, D_FF), lambda j, te_ref: (j, 0)),
    )
    return pl.pallas_call(
        _gu_body,
        grid_spec=gs,
        out_shape=jax.ShapeDtypeStruct((NPAD, D_FF), jnp.float32),
    )(te, xs, gup)


def _dn_body(te_ref, a_ref, w_ref, y_ref):
    del te_ref
    y_ref[...] = jnp.dot(a_ref[...], w_ref[0], preferred_element_type=jnp.float32)


def _grouped_dn(te, act, dnw):
    gs = pltpu.PrefetchScalarGridSpec(
        num_scalar_prefetch=1,
        grid=(NT,),
        in_specs=[
            pl.BlockSpec((BM, D_FF), lambda j, te_ref: (j, 0)),
            pl.BlockSpec((1, D_FF, D_MODEL), lambda j, te_ref: (te_ref[j], 0, 0)),
        ],
        out_specs=pl.BlockSpec((BM, D_MODEL), lambda j, te_ref: (j, 0)),
    )
    return pl.pallas_call(
        _dn_body,
        grid_spec=gs,
        out_shape=jax.ShapeDtypeStruct((NPAD, D_MODEL), jnp.float32),
    )(te, act, dnw)


def _sgu_body(x_ref, w_ref, act_ref):
    gu = jnp.dot(x_ref[...], w_ref[...], preferred_element_type=jnp.float32)
    g = gu[:, :D_FF]
    u = gu[:, D_FF:]
    act_ref[...] = g * jax.nn.sigmoid(g) * u


def _shared_gu(hidden, sguw):
    return pl.pallas_call(
        _sgu_body,
        grid=(T // TB,),
        in_specs=[
            pl.BlockSpec((TB, D_MODEL), lambda i: (i, 0)),
            pl.BlockSpec((D_MODEL, 2 * D_FF), lambda i: (0, 0)),
        ],
        out_specs=pl.BlockSpec((TB, D_FF), lambda i: (i, 0)),
        out_shape=jax.ShapeDtypeStruct((T, D_FF), jnp.float32),
    )(hidden, sguw)


def _comb_body(a_ref, w_ref, sg_ref, g0_ref, g1_ref, o_ref):
    y = jnp.dot(a_ref[...], w_ref[...], preferred_element_type=jnp.float32)
    o_ref[...] = g0_ref[...] + g1_ref[...] + y * sg_ref[...]


def _combine(acts, sdnw, sg, g0, g1):
    return pl.pallas_call(
        _comb_body,
        grid=(T // TB,),
        in_specs=[
            pl.BlockSpec((TB, D_FF), lambda i: (i, 0)),
            pl.BlockSpec((D_FF, D_MODEL), lambda i: (0, 0)),
            pl.BlockSpec((TB, 1), lambda i: (i, 0)),
            pl.BlockSpec((TB, D_MODEL), lambda i: (i, 0)),
            pl.BlockSpec((TB, D_MODEL), lambda i: (i, 0)),
        ],
        out_specs=pl.BlockSpec((TB, D_MODEL), lambda i: (i, 0)),
        out_shape=jax.ShapeDtypeStruct((T, D_MODEL), jnp.float32),
    )(acts, sdnw, sg, g0, g1)


def kernel(hidden_states, router_w, gate_up_w, down_w, shared_gate_up_w,
           shared_down_w, shared_gate_w):
    sgw_row = shared_gate_w.reshape(1, D_MODEL)
    i1, i2, h1, h2, sg = _router(hidden_states, router_w, sgw_row)
    p1, p2, te = _sortmeta(i1, i2)
    p1r = p1.reshape(T // CH, CH)
    p2r = p2.reshape(T // CH, CH)
    te_flat = te.reshape(NTE)
    xs = _sc_dispatch(h1, h2, p1r, p2r)
    act = _grouped_gu(te_flat, xs, gate_up_w)
    ys = _grouped_dn(te_flat, act, down_w)
    acts = _shared_gu(hidden_states, shared_gate_up_w)
    g0, g1 = _sc_combine_fetch(ys, p1r, p2r)
    return _combine(acts, shared_down_w, sg, g0, g1)
